# baseline (speedup 1.0000x reference)
import jax
import jax.numpy as jnp
from jax.experimental import pallas as pl
from jax.experimental.pallas import tpu as pltpu

CV = 1024
NR = 8
CR = 64
B = 4096


def _dot(a, b):
    return jax.lax.dot_general(a, b, (((1,), (0,)), ((), ())),
                               preferred_element_type=jnp.float32)


def _dot_t(a, b):
    return jax.lax.dot_general(a, b, (((1,), (1,)), ((), ())),
                               preferred_element_type=jnp.float32)


def _dot_fast(a, b):
    return jax.lax.dot_general(a, b, (((1,), (0,)), ((), ())),
                               preferred_element_type=jnp.float32,
                               precision=jax.lax.Precision.DEFAULT)


_BF = jnp.bfloat16


def _main_kernel(o1_ref, o2_ref, opv_ref,
                 eod_W1_ref, eod_b1_ref, eop_W1_ref, eop_b1_ref,
                 eod_W2_ref, eod_b2_ref, eop_W2_ref, eop_b2_ref,
                 st_W1_ref, st_b1_ref, st_W2_ref, st_b2_ref,
                 rule_body_ref,
                 s1_q_W_ref, s1_q_b_ref, s1_k_W_ref, s1_k_b_ref,
                 s2_q_W_ref, s2_q_b_ref, s2_k_W_ref, s2_k_b_ref,
                 dec_b1_ref, dec_W2_ref, dec_b2_ref,
                 st_W2_bf_ref, dec_W1_bf_ref,
                 rh_W1_ref, rh_b1_ref, rh_W2_ref, rh_b2_ref,
                 out_ref,
                 gp_ref, gc_ref, idxr_ref, acc_ref):
    r = pl.program_id(0)

    @pl.when(r == 0)
    def _route():
        st_W1b = st_W1_ref[CV:, :]
        st_b1 = st_b1_ref[...]
        st_W2 = st_W2_ref[...]
        st_b2 = st_b2_ref[...]

        M_od = _dot(eod_W2_ref[...], st_W1b)
        c_od = _dot(eod_b2_ref[...], st_W1b) + st_b1
        M_op = _dot(eop_W2_ref[...], st_W1b)
        c_op = _dot(eop_b2_ref[...], st_W1b) + st_b1

        q1 = _dot(rule_body_ref[...], s1_q_W_ref[...]) + s1_q_b_ref[...]
        K1 = _dot(st_W2, s1_k_W_ref[...])
        d1 = _dot(st_b2, s1_k_W_ref[...]) + s1_k_b_ref[...]
        P1 = _dot_t(K1, q1)
        r1 = _dot_t(d1, q1)
        q2 = _dot(rule_body_ref[...], s2_q_W_ref[...]) + s2_q_b_ref[...]
        K2 = _dot(st_W2, s2_k_W_ref[...])
        d2 = _dot(st_b2, s2_k_W_ref[...]) + s2_k_b_ref[...]
        P2 = _dot_t(K2, q2)
        r2 = _dot_t(d2, q2)

        o1 = o1_ref[...]
        o2 = o2_ref[...]
        op_idx = opv_ref[...].astype(jnp.int32)

        w0 = eod_W1_ref[0:1, :]
        w1 = eod_W1_ref[1:2, :]
        b1 = eod_b1_ref[...]
        u1 = jax.nn.relu(o1 * w0 + b1)
        u2 = jax.nn.relu(o2 * w0 + w1 + b1)
        p0 = eop_W1_ref[0:1, :]
        p1 = eop_W1_ref[1:2, :]
        p2 = eop_W1_ref[2:3, :]
        u3 = jax.nn.relu(
            jnp.where(op_idx == 0, p0, jnp.where(op_idx == 1, p1, p2))
            + eop_b1_ref[...])

        g1 = jax.nn.relu(_dot(u1, M_od) + c_od)
        g2 = jax.nn.relu(_dot(u2, M_od) + c_od)
        g3 = jax.nn.relu(_dot(u3, M_op) + c_op)

        l1 = _dot(g1, P1) + r1
        l2 = _dot(g2, P1) + r1
        l3 = _dot(g3, P1) + r1
        m = jnp.maximum(jnp.maximum(jnp.max(l1, axis=1, keepdims=True),
                                    jnp.max(l2, axis=1, keepdims=True)),
                        jnp.max(l3, axis=1, keepdims=True))
        ncol = jax.lax.broadcasted_iota(jnp.int32, l1.shape, 1)
        big = jnp.int32(1 << 20)
        cand1 = jnp.min(jnp.where(l1 == m, 3 * ncol + 0, big), axis=1,
                        keepdims=True)
        cand2 = jnp.min(jnp.where(l2 == m, 3 * ncol + 1, big), axis=1,
                        keepdims=True)
        cand3 = jnp.min(jnp.where(l3 == m, 3 * ncol + 2, big), axis=1,
                        keepdims=True)
        idx0 = jnp.minimum(jnp.minimum(cand1, cand2), cand3)
        idx_r = idx0 // 3
        idx_p = idx0 - 3 * idx_r

        a1 = _dot(g1, P2) + r2
        a2 = _dot(g2, P2) + r2
        onehot_r = (ncol == idx_r)
        a1s = jnp.sum(jnp.where(onehot_r, a1, 0.0), axis=1, keepdims=True)
        a2s = jnp.sum(jnp.where(onehot_r, a2, 0.0), axis=1, keepdims=True)
        idx_c = jnp.where(a1s >= a2s, 0, 1)

        g_p = jnp.where(idx_p == 0, g1, jnp.where(idx_p == 1, g2, g3))
        g_c = jnp.where(idx_c == 0, g1, g2)
        gp_ref[...] = g_p.astype(_BF)
        gc_ref[...] = g_c.astype(_BF)
        idxr_ref[...] = idx_r.astype(jnp.float32)

    st_W2_bf = st_W2_bf_ref[...]
    st_b2_bf = st_b2_ref[...].astype(_BF)
    W1_top = rh_W1_ref[0, :CV, :]
    W1_bot = rh_W1_ref[0, CV:, :]
    A_r = _dot_fast(st_W2_bf, W1_top).astype(_BF)
    C_r = _dot_fast(st_W2_bf, W1_bot).astype(_BF)
    e_r = (_dot_fast(st_b2_bf, W1_top) + _dot_fast(st_b2_bf, W1_bot)
           + rh_b1_ref[0])
    D_r = _dot_fast(rh_W2_ref[0], dec_W1_bf_ref[...]).astype(_BF)
    f_r = _dot_fast(rh_b2_ref[0].astype(_BF), dec_W1_bf_ref[...])

    h = jax.nn.relu(_dot_fast(gp_ref[...], A_r)
                    + _dot_fast(gc_ref[...], C_r) + e_r)
    mask = idxr_ref[...] == r.astype(jnp.float32)
    contrib = jnp.where(mask, _dot_fast(h.astype(_BF), D_r) + f_r, 0.0)
    prev = jnp.where(r == 0, 0.0, acc_ref[...])
    acc = prev + contrib
    acc_ref[...] = acc

    @pl.when(r == NR - 1)
    def _decode():
        t = jax.nn.relu(acc + dec_b1_ref[...])
        out_ref[...] = _dot(t, dec_W2_ref[...]) + dec_b2_ref[...]


@jax.jit
def kernel(operand1, operand2, operator, enc_od_W1, enc_od_b1, enc_od_W2,
           enc_od_b2, enc_op_W1, enc_op_b1, enc_op_W2, enc_op_b2, dec_W1,
           dec_b1, dec_W2, dec_b2, st_W1, st_b1, st_W2, st_b2, rule_body,
           rh_W1, rh_b1, rh_W2, rh_b2, s1_q_W, s1_q_b, s1_k_W, s1_k_b,
           s2_q_W, s2_q_b, s2_k_W, s2_k_b):
    row = lambda v: v.reshape(1, -1)
    f32 = jnp.float32

    full = lambda a: pl.BlockSpec(a.shape, lambda r: (0,) * a.ndim)
    stream3 = lambda a: pl.BlockSpec((1,) + a.shape[1:], lambda r: (r, 0, 0))

    rh_W1_bf = rh_W1.astype(_BF)
    rh_W2_bf = rh_W2.astype(_BF)
    st_W2_bf = st_W2.astype(_BF)
    dec_W1_bf = dec_W1.astype(_BF)
    rh_b1_3 = rh_b1.reshape(NR, 1, 128)
    rh_b2_3 = rh_b2.reshape(NR, 1, CV)
    consts = [operand1.reshape(B, 1), operand2.reshape(B, 1),
              operator.reshape(B, 1),
              enc_od_W1, row(enc_od_b1), enc_op_W1, row(enc_op_b1),
              enc_od_W2, row(enc_od_b2), enc_op_W2, row(enc_op_b2),
              st_W1, row(st_b1), st_W2, row(st_b2),
              rule_body,
              s1_q_W, row(s1_q_b), s1_k_W, row(s1_k_b),
              s2_q_W, row(s2_q_b), s2_k_W, row(s2_k_b),
              row(dec_b1), dec_W2, dec_b2.reshape(1, 1),
              st_W2_bf, dec_W1_bf]

    out = pl.pallas_call(
        _main_kernel,
        grid=(NR,),
        in_specs=[full(a) for a in consts]
        + [stream3(rh_W1_bf), stream3(rh_b1_3),
           stream3(rh_W2_bf), stream3(rh_b2_3)],
        out_specs=pl.BlockSpec((B, 1), lambda r: (0, 0)),
        out_shape=jax.ShapeDtypeStruct((B, 1), f32),
        scratch_shapes=[
            pltpu.VMEM((B, 64), _BF),
            pltpu.VMEM((B, 64), _BF),
            pltpu.VMEM((B, 1), f32),
            pltpu.VMEM((B, 64), f32),
        ],
    )(*consts, rh_W1_bf, rh_b1_3, rh_W2_bf, rh_b2_3)
    return out.reshape(B)


# speedup vs baseline: 1.2629x; 1.2629x over previous
import jax
import jax.numpy as jnp
from jax.experimental import pallas as pl
from jax.experimental.pallas import tpu as pltpu

CV = 1024
NR = 8
CR = 64
B = 4096


def _dot(a, b):
    return jax.lax.dot_general(a, b, (((1,), (0,)), ((), ())),
                               preferred_element_type=jnp.float32)


def _dot_t(a, b):
    return jax.lax.dot_general(a, b, (((1,), (1,)), ((), ())),
                               preferred_element_type=jnp.float32)


def _dot_fast(a, b):
    return jax.lax.dot_general(a, b, (((1,), (0,)), ((), ())),
                               preferred_element_type=jnp.float32,
                               precision=jax.lax.Precision.DEFAULT)


def _main_kernel(o1_ref, o2_ref, opv_ref,
                 eod_W1_ref, eod_b1_ref, eop_W1_ref, eop_b1_ref,
                 eod_W2_ref, eod_b2_ref, eop_W2_ref, eop_b2_ref,
                 st_W1_ref, st_b1_ref, st_W2_ref, st_b2_ref,
                 rule_body_ref,
                 s1_q_W_ref, s1_q_b_ref, s1_k_W_ref, s1_k_b_ref,
                 s2_q_W_ref, s2_q_b_ref, s2_k_W_ref, s2_k_b_ref,
                 dec_W1_ref, dec_b1_ref, dec_W2_ref, dec_b2_ref,
                 rh_W1_ref, rh_b1_ref, rh_W2_ref, rh_b2_ref,
                 out_ref,
                 M_od_ref, c_od_ref, M_op_ref, c_op_ref,
                 P1_ref, r1_ref, P2_ref, r2_ref,
                 A_ref, C_ref, e_ref, D_ref, f_ref):
    step = pl.program_id(0)

    @pl.when(step == 0)
    def _fuse():
        st_W1b = st_W1_ref[CV:, :]
        st_b1 = st_b1_ref[...]
        st_W2 = st_W2_ref[...]
        st_b2 = st_b2_ref[...]

        M_od_ref[...] = _dot(eod_W2_ref[...], st_W1b)
        c_od_ref[...] = _dot(eod_b2_ref[...], st_W1b) + st_b1
        M_op_ref[...] = _dot(eop_W2_ref[...], st_W1b)
        c_op_ref[...] = _dot(eop_b2_ref[...], st_W1b) + st_b1

        q1 = _dot(rule_body_ref[...], s1_q_W_ref[...]) + s1_q_b_ref[...]
        K1 = _dot(st_W2, s1_k_W_ref[...])
        d1 = _dot(st_b2, s1_k_W_ref[...]) + s1_k_b_ref[...]
        P1_ref[...] = _dot_t(K1, q1)
        r1_ref[...] = _dot_t(d1, q1)
        q2 = _dot(rule_body_ref[...], s2_q_W_ref[...]) + s2_q_b_ref[...]
        K2 = _dot(st_W2, s2_k_W_ref[...])
        d2 = _dot(st_b2, s2_k_W_ref[...]) + s2_k_b_ref[...]
        P2_ref[...] = _dot_t(K2, q2)
        r2_ref[...] = _dot_t(d2, q2)

        for r in range(NR):
            W1_top = rh_W1_ref[r, :CV, :]
            W1_bot = rh_W1_ref[r, CV:, :]
            A_ref[:, r * 128:(r + 1) * 128] = _dot(st_W2, W1_top)
            C_ref[:, r * 128:(r + 1) * 128] = _dot(st_W2, W1_bot)
            e_ref[:, r * 128:(r + 1) * 128] = (
                _dot(st_b2, W1_top) + _dot(st_b2, W1_bot)
                + rh_b1_ref[r:r + 1, :])
            D_ref[r * 128:(r + 1) * 128, :] = _dot(rh_W2_ref[r],
                                                   dec_W1_ref[...])
        f_ref[...] = _dot(rh_b2_ref[...], dec_W1_ref[...]) + dec_b1_ref[...]

    @pl.when(step == 1)
    def _main():
        o1 = o1_ref[...]
        o2 = o2_ref[...]
        op_idx = opv_ref[...].astype(jnp.int32)

        w0 = eod_W1_ref[0:1, :]
        w1 = eod_W1_ref[1:2, :]
        b1 = eod_b1_ref[...]
        u1 = jax.nn.relu(o1 * w0 + b1)
        u2 = jax.nn.relu(o2 * w0 + w1 + b1)
        p0 = eop_W1_ref[0:1, :]
        p1 = eop_W1_ref[1:2, :]
        p2 = eop_W1_ref[2:3, :]
        u3 = jax.nn.relu(
            jnp.where(op_idx == 0, p0, jnp.where(op_idx == 1, p1, p2))
            + eop_b1_ref[...])

        g1 = jax.nn.relu(_dot(u1, M_od_ref[...]) + c_od_ref[...])
        g2 = jax.nn.relu(_dot(u2, M_od_ref[...]) + c_od_ref[...])
        g3 = jax.nn.relu(_dot(u3, M_op_ref[...]) + c_op_ref[...])

        P1 = P1_ref[...]
        r1 = r1_ref[...]
        l1 = _dot(g1, P1) + r1
        l2 = _dot(g2, P1) + r1
        l3 = _dot(g3, P1) + r1
        m = jnp.maximum(jnp.maximum(jnp.max(l1, axis=1, keepdims=True),
                                    jnp.max(l2, axis=1, keepdims=True)),
                        jnp.max(l3, axis=1, keepdims=True))
        ncol = jax.lax.broadcasted_iota(jnp.int32, l1.shape, 1)
        big = jnp.int32(1 << 20)
        cand1 = jnp.min(jnp.where(l1 == m, 3 * ncol + 0, big), axis=1,
                        keepdims=True)
        cand2 = jnp.min(jnp.where(l2 == m, 3 * ncol + 1, big), axis=1,
                        keepdims=True)
        cand3 = jnp.min(jnp.where(l3 == m, 3 * ncol + 2, big), axis=1,
                        keepdims=True)
        idx0 = jnp.minimum(jnp.minimum(cand1, cand2), cand3)
        idx_r = idx0 // 3
        idx_p = idx0 - 3 * idx_r

        a1 = _dot(g1, P2_ref[...]) + r2_ref[...]
        a2 = _dot(g2, P2_ref[...]) + r2_ref[...]
        onehot_r = (ncol == idx_r)
        a1s = jnp.sum(jnp.where(onehot_r, a1, 0.0), axis=1, keepdims=True)
        a2s = jnp.sum(jnp.where(onehot_r, a2, 0.0), axis=1, keepdims=True)
        idx_c = jnp.where(a1s >= a2s, 0, 1)

        g_p = jnp.where(idx_p == 0, g1, jnp.where(idx_p == 1, g2, g3))
        g_c = jnp.where(idx_c == 0, g1, g2)

        h = jax.nn.relu(_dot_fast(g_p, A_ref[...]) + _dot_fast(g_c, C_ref[...])
                        + e_ref[...])
        colr = jax.lax.broadcasted_iota(jnp.int32, h.shape, 1) // 128
        hm = jnp.where(colr == idx_r, h, 0.0)
        f_sel = _dot(onehot_r.astype(jnp.float32), f_ref[...])
        t = jax.nn.relu(_dot_fast(hm, D_ref[...]) + f_sel)
        out_ref[...] = _dot(t, dec_W2_ref[...]) + dec_b2_ref[...]


@jax.jit
def kernel(operand1, operand2, operator, enc_od_W1, enc_od_b1, enc_od_W2,
           enc_od_b2, enc_op_W1, enc_op_b1, enc_op_W2, enc_op_b2, dec_W1,
           dec_b1, dec_W2, dec_b2, st_W1, st_b1, st_W2, st_b2, rule_body,
           rh_W1, rh_b1, rh_W2, rh_b2, s1_q_W, s1_q_b, s1_k_W, s1_k_b,
           s2_q_W, s2_q_b, s2_k_W, s2_k_b):
    row = lambda v: v.reshape(1, -1)
    f32 = jnp.float32

    full = lambda a: pl.BlockSpec(a.shape, lambda s: (0,) * a.ndim)

    args = [operand1.reshape(B, 1), operand2.reshape(B, 1),
            operator.reshape(B, 1),
            enc_od_W1, row(enc_od_b1), enc_op_W1, row(enc_op_b1),
            enc_od_W2, row(enc_od_b2), enc_op_W2, row(enc_op_b2),
            st_W1, row(st_b1), st_W2, row(st_b2),
            rule_body,
            s1_q_W, row(s1_q_b), s1_k_W, row(s1_k_b),
            s2_q_W, row(s2_q_b), s2_k_W, row(s2_k_b),
            dec_W1, row(dec_b1), dec_W2, dec_b2.reshape(1, 1),
            rh_W1, rh_b1, rh_W2, rh_b2]

    out = pl.pallas_call(
        _main_kernel,
        grid=(2,),
        in_specs=[full(a) for a in args],
        out_specs=pl.BlockSpec((B, 1), lambda s: (0, 0)),
        out_shape=jax.ShapeDtypeStruct((B, 1), f32),
        scratch_shapes=[
            pltpu.VMEM((64, 64), f32),
            pltpu.VMEM((1, 64), f32),
            pltpu.VMEM((64, 64), f32),
            pltpu.VMEM((1, 64), f32),
            pltpu.VMEM((64, NR), f32),
            pltpu.VMEM((1, NR), f32),
            pltpu.VMEM((64, NR), f32),
            pltpu.VMEM((1, NR), f32),
            pltpu.VMEM((64, NR * 128), f32),
            pltpu.VMEM((64, NR * 128), f32),
            pltpu.VMEM((1, NR * 128), f32),
            pltpu.VMEM((NR * 128, 64), f32),
            pltpu.VMEM((NR, 64), f32),
        ],
    )(*args)
    return out.reshape(B)
